# baseline (speedup 1.0000x reference)
import jax
import jax.numpy as jnp
from jax import lax
from jax.experimental import pallas as pl
from jax.experimental.pallas import tpu as pltpu
from jax.experimental.pallas import tpu_sc as plsc

_B = 16384
_D_NUM = 13
_N_CAT = 26
_D_TOKEN = 128
_NW = 32
_RPW = _B // _NW
_R = 128
_NCHUNK = _RPW // _R
_LANES = 8


def _sc_body(xnum_hbm, xcatt_hbm, emb_hbm, w_hbm, b_hbm, offs_hbm, out_hbm,
             w_v, b_v, offs_v, xnum_v, idx_v, gath_v, col_v, sem):
    cid = lax.axis_index("c")
    sid = lax.axis_index("s")
    wid = sid * 2 + cid
    row0w = wid * _RPW

    pltpu.sync_copy(w_hbm, w_v)
    pltpu.sync_copy(b_hbm, b_v)
    pltpu.sync_copy(offs_hbm, offs_v)

    @pl.loop(0, _NCHUNK)
    def _chunk(ch):
        row0 = row0w + ch * _R

        @pl.loop(0, _D_NUM)
        def _num_col(d):
            pltpu.sync_copy(xnum_hbm.at[d, pl.ds(row0, _R)],
                            xnum_v.at[pl.ds(0, _R)])

            @pl.loop(0, _R)
            def _num_row(r):
                xs = xnum_v[pl.ds(r, 16)][0]
                for c in range(_LANES):
                    col_v[r, 0, pl.ds(16 * c, 16)] = (
                        xs * w_v[d, pl.ds(16 * c, 16)]
                        + b_v[d, pl.ds(16 * c, 16)])
            pltpu.sync_copy(col_v, out_hbm.at[pl.ds(row0, _R), pl.ds(d, 1), :])

        @pl.loop(0, _N_CAT)
        def _cat_col(j):
            pltpu.sync_copy(xcatt_hbm.at[j, pl.ds(row0, _R)], idx_v)
            off = offs_v[pl.ds(j, 16)][0]
            for c in range(_LANES):
                idx_v[pl.ds(16 * c, 16)] = idx_v[pl.ds(16 * c, 16)] + off
            pltpu.async_copy(emb_hbm.at[idx_v], gath_v, sem).wait()

            @pl.loop(0, _R)
            def _cat_row(r):
                for c in range(_LANES):
                    col_v[r, 0, pl.ds(16 * c, 16)] = (
                        gath_v[r, pl.ds(16 * c, 16)]
                        + b_v[13 + j, pl.ds(16 * c, 16)])
            pltpu.sync_copy(col_v,
                            out_hbm.at[pl.ds(row0, _R), pl.ds(13 + j, 1), :])


def kernel(x_num, x_cat, emb_table, weight, bias, category_offsets):
    x_num_t = x_num.T
    x_cat_t = x_cat.T
    offs_pad = jnp.zeros((48,), jnp.int32).at[:_N_CAT].set(category_offsets)
    mesh = plsc.VectorSubcoreMesh(core_axis_name="c", subcore_axis_name="s")
    f = pl.kernel(
        _sc_body,
        out_type=jax.ShapeDtypeStruct((_B, _D_NUM + _N_CAT, _D_TOKEN),
                                      jnp.float32),
        mesh=mesh,
        scratch_types=[
            pltpu.VMEM((_D_NUM, _D_TOKEN), jnp.float32),
            pltpu.VMEM((_D_NUM + _N_CAT, _D_TOKEN), jnp.float32),
            pltpu.VMEM((48,), jnp.int32),
            pltpu.VMEM((_R + 16,), jnp.float32),
            pltpu.VMEM((_R,), jnp.int32),
            pltpu.VMEM((_R, _D_TOKEN), jnp.float32),
            pltpu.VMEM((_R, 1, _D_TOKEN), jnp.float32),
            pltpu.SemaphoreType.DMA,
        ],
    )
    return f(x_num_t, x_cat_t, emb_table, weight, bias, offs_pad)
